# baseline (speedup 1.0000x reference)
import functools

import jax
import jax.numpy as jnp
from jax import lax
from jax.experimental import pallas as pl
from jax.experimental.pallas import tpu as pltpu
from jax.experimental.pallas import tpu_sc as plsc


def _leaky(v):
    return jnp.where(v >= 0, v, v * 0.01)


def _ae_mlp_body(x_ref, xn_ref, w_ref, b_ref, w2_ref, b2_ref, w3_ref, b3_ref,
                 o_ref, h_scr, *, nk):
    k = pl.program_id(0)
    xb = jnp.concatenate([x_ref[...], xn_ref[...]], axis=0)
    part = jnp.dot(xb, w_ref[...], preferred_element_type=jnp.float32)

    @pl.when(k == 0)
    def _():
        h_scr[...] = part

    @pl.when(k > 0)
    def _():
        h_scr[...] += part

    @pl.when(k == nk - 1)
    def _():
        h1 = _leaky(h_scr[...] + b_ref[...])
        h2 = _leaky(jnp.dot(h1, w2_ref[...],
                            preferred_element_type=jnp.float32) + b2_ref[...])
        o_ref[...] = jnp.dot(h2, w3_ref[...],
                             preferred_element_type=jnp.float32) + b3_ref[...]


def _ae_mlp(x, xn, w, b, w2, b2, w3, b3, bk=2048):
    M2, K = x.shape
    _, N = w.shape
    _, N3 = w3.shape
    nk = K // bk
    return pl.pallas_call(
        functools.partial(_ae_mlp_body, nk=nk),
        grid=(nk,),
        in_specs=[
            pl.BlockSpec((M2, bk), lambda k: (0, k)),
            pl.BlockSpec((M2, bk), lambda k: (0, k)),
            pl.BlockSpec((bk, N), lambda k: (k, 0)),
            pl.BlockSpec((1, N), lambda k: (0, 0)),
            pl.BlockSpec((N, N), lambda k: (0, 0)),
            pl.BlockSpec((1, N), lambda k: (0, 0)),
            pl.BlockSpec((N, N3), lambda k: (0, 0)),
            pl.BlockSpec((1, N3), lambda k: (0, 0)),
        ],
        out_specs=pl.BlockSpec((2 * M2, N3), lambda k: (0, 0)),
        out_shape=jax.ShapeDtypeStruct((2 * M2, N3), jnp.float32),
        scratch_shapes=[pltpu.VMEM((2 * M2, N), jnp.float32)],
        compiler_params=pltpu.CompilerParams(
            dimension_semantics=("arbitrary",)),
    )(x, xn, w, b.reshape(1, N), w2, b2.reshape(1, N), w3, b3.reshape(1, N3))


def _ae_vq_body(z_ref, e_ref, zst_ref, diff_ref):
    z = z_ref[...]
    cols = []
    for g in range(16):
        zf = z[:, g * 32:(g + 1) * 32]
        eg = e_ref[g]
        s = lax.dot_general(zf, eg, (((1,), (1,)), ((), ())),
                            preferred_element_type=jnp.float32)
        zn = jnp.sum(zf * zf, axis=1, keepdims=True)
        en = jnp.sum(eg * eg, axis=1)
        dist = zn - 2.0 * s + en[None, :]
        m = jnp.min(dist, axis=1, keepdims=True)
        ii = lax.broadcasted_iota(jnp.int32, dist.shape, 1)
        arg = jnp.min(jnp.where(dist == m, ii, jnp.int32(2 ** 30)),
                      axis=1, keepdims=True)
        onehot = (ii == arg).astype(jnp.float32)
        cols.append(lax.dot_general(
            onehot, eg, (((1,), (0,)), ((), ())),
            precision=lax.Precision.HIGHEST,
            preferred_element_type=jnp.float32))
    zq = jnp.concatenate(cols, axis=1)
    d = zq - z
    sq = d * d
    h = z.shape[0] // 2
    diff_ref[...] = jnp.stack(
        [jnp.sum(sq[:h]), jnp.sum(sq[h:])]).reshape(2, 1)
    zst_ref[...] = z + d


def _ae_vq(z, embed):
    return pl.pallas_call(
        _ae_vq_body,
        out_shape=[
            jax.ShapeDtypeStruct(z.shape, jnp.float32),
            jax.ShapeDtypeStruct((2, 1), jnp.float32),
        ],
    )(z, embed)


def _encvq_body(zst_ref, w1_ref, b1_ref, w2_ref, b2_ref, q_ref,
                xe_ref, ind_ref, bd, bi, zn_scr):
    c = pl.program_id(0)
    nc = pl.num_programs(0)

    @pl.when(c == 0)
    def _():
        e1 = _leaky(jnp.dot(zst_ref[...], w1_ref[...],
                            preferred_element_type=jnp.float32) + b1_ref[...])
        v = jnp.dot(e1, w2_ref[...],
                    preferred_element_type=jnp.float32) + b2_ref[...]
        xe_ref[...] = v
        zn_scr[...] = jnp.sum(v * v, axis=1, keepdims=True)

    xe = xe_ref[...]
    qb = q_ref[...]
    s = lax.dot_general(xe, qb, (((1,), (1,)), ((), ())),
                        preferred_element_type=jnp.float32)
    zn = zn_scr[...]
    en = jnp.sum(qb * qb, axis=1)
    dist = zn - 2.0 * s + en[None, :]
    m = jnp.min(dist, axis=1, keepdims=True)
    ii = lax.broadcasted_iota(jnp.int32, dist.shape, 1)
    arg = (jnp.min(jnp.where(dist == m, ii, jnp.int32(2 ** 30)),
                   axis=1, keepdims=True) + c * q_ref.shape[0])

    @pl.when(c == 0)
    def _():
        bd[...] = m
        bi[...] = arg

    @pl.when(c > 0)
    def _():
        better = m < bd[...]
        bd[...] = jnp.where(better, m, bd[...])
        bi[...] = jnp.where(better, arg, bi[...])

    @pl.when(c == nc - 1)
    def _():
        ind_ref[...] = bi[...]


def _enc_vq(zst, w1, b1, w2, b2, q0, bc=1024):
    M, K = zst.shape
    _, N1 = w1.shape
    _, N = w2.shape
    nc = q0.shape[0] // bc
    return pl.pallas_call(
        _encvq_body,
        grid=(nc,),
        in_specs=[
            pl.BlockSpec((M, K), lambda c: (0, 0)),
            pl.BlockSpec((K, N1), lambda c: (0, 0)),
            pl.BlockSpec((1, N1), lambda c: (0, 0)),
            pl.BlockSpec((N1, N), lambda c: (0, 0)),
            pl.BlockSpec((1, N), lambda c: (0, 0)),
            pl.BlockSpec((bc, q0.shape[1]), lambda c: (c, 0)),
        ],
        out_specs=[
            pl.BlockSpec((M, N), lambda c: (0, 0)),
            pl.BlockSpec((M, 1), lambda c: (0, 0)),
        ],
        out_shape=[
            jax.ShapeDtypeStruct((M, N), jnp.float32),
            jax.ShapeDtypeStruct((M, 1), jnp.int32),
        ],
        scratch_shapes=[
            pltpu.VMEM((M, 1), jnp.float32),
            pltpu.VMEM((M, 1), jnp.int32),
            pltpu.VMEM((M, 1), jnp.float32),
        ],
        compiler_params=pltpu.CompilerParams(
            dimension_semantics=("arbitrary",)),
    )(zst, w1, b1.reshape(1, N1), w2, b2.reshape(1, N), q0)


def _decloss_body(zst_ref, wa_ref, ba_ref, w_ref, b_ref, x_ref, xn_ref,
                  o_ref, d_scr):
    n = pl.program_id(0)

    @pl.when(n == 0)
    def _():
        o_ref[...] = jnp.zeros_like(o_ref)
        d_scr[...] = _leaky(jnp.dot(zst_ref[...], wa_ref[...],
                                    preferred_element_type=jnp.float32)
                            + ba_ref[...])

    rec = jnp.dot(d_scr[...], w_ref[...],
                  preferred_element_type=jnp.float32) + b_ref[...]
    M2 = x_ref.shape[0]
    e1 = rec[:M2] - x_ref[...]
    e2 = rec[M2:] - xn_ref[...]
    s = jnp.stack([jnp.sum(e1 * e1), jnp.sum(e2 * e2)]).reshape(2, 1)
    o_ref[...] += s


def _dec_loss(zst, wa, ba, w, b, x, xn, bn=512):
    M, K = zst.shape
    _, Na = wa.shape
    _, N = w.shape
    M2 = x.shape[0]
    return pl.pallas_call(
        _decloss_body,
        grid=(N // bn,),
        in_specs=[
            pl.BlockSpec((M, K), lambda n: (0, 0)),
            pl.BlockSpec((K, Na), lambda n: (0, 0)),
            pl.BlockSpec((1, Na), lambda n: (0, 0)),
            pl.BlockSpec((Na, bn), lambda n: (0, n)),
            pl.BlockSpec((1, bn), lambda n: (0, n)),
            pl.BlockSpec((M2, bn), lambda n: (0, n)),
            pl.BlockSpec((M2, bn), lambda n: (0, n)),
        ],
        out_specs=pl.BlockSpec((2, 1), lambda n: (0, 0)),
        out_shape=jax.ShapeDtypeStruct((2, 1), jnp.float32),
        scratch_shapes=[pltpu.VMEM((M, Na), jnp.float32)],
        compiler_params=pltpu.CompilerParams(
            dimension_semantics=("arbitrary",)),
    )(zst, wa, ba.reshape(1, Na), w, b.reshape(1, N), x, xn)


def _outhead_body(dq_ref, zq_ref, xe_ref, off_ref, w1_ref, b1_ref,
                  w2_ref, b2_ref, w3_ref, b3_ref,
                  out_ref, z1_ref, z2_ref, el_ref):
    zq = zq_ref[...]
    xe = xe_ref[...]
    d = zq - xe
    zst = xe + d
    dq = dq_ref[0] != 0
    zo = jnp.where(dq, zst, xe)
    M2 = zo.shape[0] // 2
    z1 = zo[:M2]
    z2 = zo[M2:]

    z1_ref[...] = z1
    z2_ref[...] = z2
    sq = d * d
    el_ref[...] = jnp.stack(
        [jnp.sum(sq[:M2]), jnp.sum(sq[M2:])]).reshape(2, 1)

    w1 = w1_ref[...]
    K = zo.shape[1]
    acc = jnp.dot(z1, w1[:K], preferred_element_type=jnp.float32)
    acc += jnp.dot(z2, w1[K:2 * K], preferred_element_type=jnp.float32)
    acc += jnp.dot(off_ref[...], w1[2 * K:],
                   preferred_element_type=jnp.float32)
    h1 = _leaky(acc + b1_ref[...])
    h2 = _leaky(jnp.dot(h1, w2_ref[...],
                        preferred_element_type=jnp.float32) + b2_ref[...])
    out_ref[...] = jnp.dot(h2, w3_ref[...],
                           preferred_element_type=jnp.float32) + b3_ref[...]


def _out_head(dq, zq0, xe, offs, w1, b1, w2, b2, w3, b3):
    M, K = zq0.shape
    M2 = M // 2
    K3, N1 = w1.shape
    _, N3 = w3.shape
    return pl.pallas_call(
        _outhead_body,
        in_specs=[
            pl.BlockSpec(memory_space=pltpu.SMEM),
            pl.BlockSpec((M, K), lambda: (0, 0)),
            pl.BlockSpec((M, K), lambda: (0, 0)),
            pl.BlockSpec((M2, K), lambda: (0, 0)),
            pl.BlockSpec((K3, N1), lambda: (0, 0)),
            pl.BlockSpec((1, N1), lambda: (0, 0)),
            pl.BlockSpec((N1, N1), lambda: (0, 0)),
            pl.BlockSpec((1, N1), lambda: (0, 0)),
            pl.BlockSpec((N1, N3), lambda: (0, 0)),
            pl.BlockSpec((1, N3), lambda: (0, 0)),
        ],
        out_specs=[
            pl.BlockSpec((M2, N3), lambda: (0, 0)),
            pl.BlockSpec((M2, K), lambda: (0, 0)),
            pl.BlockSpec((M2, K), lambda: (0, 0)),
            pl.BlockSpec((2, 1), lambda: (0, 0)),
        ],
        out_shape=[
            jax.ShapeDtypeStruct((M2, N3), jnp.float32),
            jax.ShapeDtypeStruct((M2, K), jnp.float32),
            jax.ShapeDtypeStruct((M2, K), jnp.float32),
            jax.ShapeDtypeStruct((2, 1), jnp.float32),
        ],
    )(dq, zq0, xe, offs, w1, b1.reshape(1, N1),
      w2, b2.reshape(1, N1), w3, b3.reshape(1, N3))


def _sc_mesh_info():
    info = plsc.get_sparse_core_info()
    return info.num_cores, info.num_subcores


def _gather_offset(otab, oidx):
    nc, ns = _sc_mesh_info()
    nw = nc * ns
    bO = oidx.shape[0] // nw
    mesh = plsc.VectorSubcoreMesh(core_axis_name="c", subcore_axis_name="s")

    @functools.partial(
        pl.kernel, mesh=mesh,
        out_type=jax.ShapeDtypeStruct((oidx.shape[0], otab.shape[1]),
                                      jnp.float32),
        scratch_types=[
            pltpu.VMEM((bO,), jnp.int32),
            pltpu.VMEM((bO, otab.shape[1]), jnp.float32),
            pltpu.SemaphoreType.DMA,
        ],
    )
    def k(otab_hbm, oidx_hbm, offs_hbm, oidx_v, orows_v, sem_o):
        wid = lax.axis_index("s") * nc + lax.axis_index("c")
        obase = wid * bO
        pltpu.sync_copy(oidx_hbm.at[pl.ds(obase, bO)], oidx_v)
        pltpu.async_copy(otab_hbm.at[oidx_v], orows_v, sem_o).wait()
        pltpu.sync_copy(orows_v, offs_hbm.at[pl.ds(obase, bO)])

    return k(otab, oidx)


def _gather_q0(tab, idx, n_chunks=4):
    nc, ns = _sc_mesh_info()
    nw = nc * ns
    bq = idx.shape[0] // nw
    ck = bq // n_chunks
    mesh = plsc.VectorSubcoreMesh(core_axis_name="c", subcore_axis_name="s")

    @functools.partial(
        pl.kernel, mesh=mesh,
        out_type=jax.ShapeDtypeStruct((idx.shape[0], tab.shape[1]),
                                      jnp.float32),
        scratch_types=(
            [pltpu.VMEM((bq,), jnp.int32),
             pltpu.VMEM((bq, tab.shape[1]), jnp.float32)]
            + [pltpu.SemaphoreType.DMA] * (n_chunks + 1)
        ),
    )
    def k(tab_hbm, idx_hbm, out_hbm, idx_v, rows_v, *sems):
        gsems, wsem = sems[:-1], sems[-1]
        wid = lax.axis_index("s") * nc + lax.axis_index("c")
        base = wid * bq
        pltpu.sync_copy(idx_hbm.at[pl.ds(base, bq)], idx_v)
        cps = []
        for ch in range(n_chunks):
            cps.append(pltpu.async_copy(
                tab_hbm.at[idx_v.at[pl.ds(ch * ck, ck)]],
                rows_v.at[pl.ds(ch * ck, ck)], gsems[ch]))
        wcps = []
        for ch in range(n_chunks):
            cps[ch].wait()
            wcps.append(pltpu.async_copy(
                rows_v.at[pl.ds(ch * ck, ck)],
                out_hbm.at[pl.ds(base + ch * ck, ck)], wsem))
        for wcp in wcps:
            wcp.wait()

    return k(tab, idx)


def kernel(x, x_next, k_offset, do_quantize, k, params):
    p = params
    B = x.shape[0]
    xf1 = x.reshape(B, -1)
    xf2 = x_next.reshape(B, -1)

    koff = k_offset.astype(jnp.int32)
    offs = _gather_offset(p['offset_table'], koff)

    zenc = _ae_mlp(xf1, xf2, p['ae_W1'], p['ae_b1'],
                   p['ae_W2'], p['ae_b2'], p['ae_W3'], p['ae_b3'])

    zst_ae, diff_ae = _ae_vq(zenc, p['ae_q_embed'])

    q0 = p['q0_embed'].reshape(8192, 512)
    xe, ind = _enc_vq(zst_ae, p['enc_W1'], p['enc_b1'],
                      p['enc_W2'], p['enc_b2'], q0)
    zq0 = _gather_q0(q0, ind.reshape(-1))

    rec_ss = _dec_loss(zst_ae, p['aed_W1'], p['aed_b1'],
                       p['aed_W2'], p['aed_b2'], xf1, xf2)

    nae = zenc.shape[0] // 2
    den_z = nae * zenc.shape[1]
    den_x = nae * xf1.shape[1]
    ae_loss_1 = rec_ss[0, 0] / den_x * 10.0 + diff_ae[0, 0] / den_z
    ae_loss_2 = rec_ss[1, 0] / den_x * 10.0 + diff_ae[1, 0] / den_z

    dq_arr = jnp.asarray(do_quantize, jnp.int32).reshape(1)
    w3 = jnp.pad(p['out_W3'], ((0, 0), (0, 118)))
    b3 = jnp.pad(p['out_b3'], (0, 118))
    out, z1, z2, el_ss = _out_head(dq_arr, zq0, xe, offs,
                                   p['out_W1'], p['out_b1'],
                                   p['out_W2'], p['out_b2'], w3, b3)
    out = out[:, :10]

    dq = do_quantize != 0
    el_1 = jnp.where(dq, el_ss[0, 0] / den_z, jnp.float32(0.0))
    el_2 = jnp.where(dq, el_ss[1, 0] / den_z, jnp.float32(0.0))

    loss = ae_loss_1 + ae_loss_2 + el_1 + el_2
    ind_1 = ind[:nae]
    ind_2 = ind[nae:]
    return (out, loss, ind_1, ind_2, z1, z2)


# speedup vs baseline: 1.0437x; 1.0437x over previous
import functools

import jax
import jax.numpy as jnp
from jax import lax
from jax.experimental import pallas as pl
from jax.experimental.pallas import tpu as pltpu
from jax.experimental.pallas import tpu_sc as plsc


def _leaky(v):
    return jnp.where(v >= 0, v, v * 0.01)


def _ae_mlp_body(x_ref, xn_ref, w_ref, b_ref, w2_ref, b2_ref, w3_ref, b3_ref,
                 o_ref, h_scr, *, nk):
    k = pl.program_id(0)
    xb = jnp.concatenate([x_ref[...], xn_ref[...]], axis=0)
    part = jnp.dot(xb, w_ref[...], preferred_element_type=jnp.float32)

    @pl.when(k == 0)
    def _():
        h_scr[...] = part

    @pl.when(k > 0)
    def _():
        h_scr[...] += part

    @pl.when(k == nk - 1)
    def _():
        h1 = _leaky(h_scr[...] + b_ref[...])
        h2 = _leaky(jnp.dot(h1, w2_ref[...],
                            preferred_element_type=jnp.float32) + b2_ref[...])
        o_ref[...] = jnp.dot(h2, w3_ref[...],
                             preferred_element_type=jnp.float32) + b3_ref[...]


def _ae_mlp(x, xn, w, b, w2, b2, w3, b3, bk=2048):
    M2, K = x.shape
    _, N = w.shape
    _, N3 = w3.shape
    nk = K // bk
    return pl.pallas_call(
        functools.partial(_ae_mlp_body, nk=nk),
        grid=(nk,),
        in_specs=[
            pl.BlockSpec((M2, bk), lambda k: (0, k)),
            pl.BlockSpec((M2, bk), lambda k: (0, k)),
            pl.BlockSpec((bk, N), lambda k: (k, 0)),
            pl.BlockSpec((1, N), lambda k: (0, 0)),
            pl.BlockSpec((N, N), lambda k: (0, 0)),
            pl.BlockSpec((1, N), lambda k: (0, 0)),
            pl.BlockSpec((N, N3), lambda k: (0, 0)),
            pl.BlockSpec((1, N3), lambda k: (0, 0)),
        ],
        out_specs=pl.BlockSpec((2 * M2, N3), lambda k: (0, 0)),
        out_shape=jax.ShapeDtypeStruct((2 * M2, N3), jnp.float32),
        scratch_shapes=[pltpu.VMEM((2 * M2, N), jnp.float32)],
        compiler_params=pltpu.CompilerParams(
            dimension_semantics=("arbitrary",)),
    )(x, xn, w, b.reshape(1, N), w2, b2.reshape(1, N), w3, b3.reshape(1, N3))


def _ae_vq_body(z_ref, e_ref, zst_ref, diff_ref):
    z = z_ref[...]
    cols = []
    for g in range(16):
        zf = z[:, g * 32:(g + 1) * 32]
        eg = e_ref[g]
        s = lax.dot_general(zf, eg, (((1,), (1,)), ((), ())),
                            preferred_element_type=jnp.float32)
        zn = jnp.sum(zf * zf, axis=1, keepdims=True)
        en = jnp.sum(eg * eg, axis=1)
        dist = zn - 2.0 * s + en[None, :]
        m = jnp.min(dist, axis=1, keepdims=True)
        ii = lax.broadcasted_iota(jnp.int32, dist.shape, 1)
        arg = jnp.min(jnp.where(dist == m, ii, jnp.int32(2 ** 30)),
                      axis=1, keepdims=True)
        onehot = (ii == arg).astype(jnp.float32)
        cols.append(lax.dot_general(
            onehot, eg, (((1,), (0,)), ((), ())),
            precision=lax.Precision.HIGHEST,
            preferred_element_type=jnp.float32))
    zq = jnp.concatenate(cols, axis=1)
    d = zq - z
    sq = d * d
    h = z.shape[0] // 2
    diff_ref[...] = jnp.stack(
        [jnp.sum(sq[:h]), jnp.sum(sq[h:])]).reshape(2, 1)
    zst_ref[...] = z + d


def _ae_vq(z, embed):
    return pl.pallas_call(
        _ae_vq_body,
        out_shape=[
            jax.ShapeDtypeStruct(z.shape, jnp.float32),
            jax.ShapeDtypeStruct((2, 1), jnp.float32),
        ],
    )(z, embed)


def _encvq_body(zst_ref, w1_ref, b1_ref, w2_ref, b2_ref, q_ref,
                xe_ref, ind_ref, bd, bi, zn_scr):
    c = pl.program_id(0)
    nc = pl.num_programs(0)

    @pl.when(c == 0)
    def _():
        e1 = _leaky(jnp.dot(zst_ref[...], w1_ref[...],
                            preferred_element_type=jnp.float32) + b1_ref[...])
        v = jnp.dot(e1, w2_ref[...],
                    preferred_element_type=jnp.float32) + b2_ref[...]
        xe_ref[...] = v
        zn_scr[...] = jnp.sum(v * v, axis=1, keepdims=True)

    xe = xe_ref[...]
    qb = q_ref[...]
    s = lax.dot_general(xe, qb, (((1,), (1,)), ((), ())),
                        preferred_element_type=jnp.float32)
    zn = zn_scr[...]
    en = jnp.sum(qb * qb, axis=1)
    dist = zn - 2.0 * s + en[None, :]
    m = jnp.min(dist, axis=1, keepdims=True)
    ii = lax.broadcasted_iota(jnp.int32, dist.shape, 1)
    arg = (jnp.min(jnp.where(dist == m, ii, jnp.int32(2 ** 30)),
                   axis=1, keepdims=True) + c * q_ref.shape[0])

    @pl.when(c == 0)
    def _():
        bd[...] = m
        bi[...] = arg

    @pl.when(c > 0)
    def _():
        better = m < bd[...]
        bd[...] = jnp.where(better, m, bd[...])
        bi[...] = jnp.where(better, arg, bi[...])

    @pl.when(c == nc - 1)
    def _():
        ind_ref[...] = bi[...]


def _enc_vq(zst, w1, b1, w2, b2, q0, bc=1024):
    M, K = zst.shape
    _, N1 = w1.shape
    _, N = w2.shape
    nc = q0.shape[0] // bc
    return pl.pallas_call(
        _encvq_body,
        grid=(nc,),
        in_specs=[
            pl.BlockSpec((M, K), lambda c: (0, 0)),
            pl.BlockSpec((K, N1), lambda c: (0, 0)),
            pl.BlockSpec((1, N1), lambda c: (0, 0)),
            pl.BlockSpec((N1, N), lambda c: (0, 0)),
            pl.BlockSpec((1, N), lambda c: (0, 0)),
            pl.BlockSpec((bc, q0.shape[1]), lambda c: (c, 0)),
        ],
        out_specs=[
            pl.BlockSpec((M, N), lambda c: (0, 0)),
            pl.BlockSpec((M, 1), lambda c: (0, 0)),
        ],
        out_shape=[
            jax.ShapeDtypeStruct((M, N), jnp.float32),
            jax.ShapeDtypeStruct((M, 1), jnp.int32),
        ],
        scratch_shapes=[
            pltpu.VMEM((M, 1), jnp.float32),
            pltpu.VMEM((M, 1), jnp.int32),
            pltpu.VMEM((M, 1), jnp.float32),
        ],
        compiler_params=pltpu.CompilerParams(
            dimension_semantics=("arbitrary",)),
    )(zst, w1, b1.reshape(1, N1), w2, b2.reshape(1, N), q0)


def _decloss_body(zst_ref, wa_ref, ba_ref, w_ref, b_ref, x_ref, xn_ref,
                  o_ref, d_scr):
    n = pl.program_id(0)

    @pl.when(n == 0)
    def _():
        o_ref[...] = jnp.zeros_like(o_ref)
        d_scr[...] = _leaky(jnp.dot(zst_ref[...], wa_ref[...],
                                    preferred_element_type=jnp.float32)
                            + ba_ref[...])

    rec = jnp.dot(d_scr[...], w_ref[...],
                  preferred_element_type=jnp.float32) + b_ref[...]
    M2 = x_ref.shape[0]
    e1 = rec[:M2] - x_ref[...]
    e2 = rec[M2:] - xn_ref[...]
    s = jnp.stack([jnp.sum(e1 * e1), jnp.sum(e2 * e2)]).reshape(2, 1)
    o_ref[...] += s


def _dec_loss(zst, wa, ba, w, b, x, xn, bn=512):
    M, K = zst.shape
    _, Na = wa.shape
    _, N = w.shape
    M2 = x.shape[0]
    return pl.pallas_call(
        _decloss_body,
        grid=(N // bn,),
        in_specs=[
            pl.BlockSpec((M, K), lambda n: (0, 0)),
            pl.BlockSpec((K, Na), lambda n: (0, 0)),
            pl.BlockSpec((1, Na), lambda n: (0, 0)),
            pl.BlockSpec((Na, bn), lambda n: (0, n)),
            pl.BlockSpec((1, bn), lambda n: (0, n)),
            pl.BlockSpec((M2, bn), lambda n: (0, n)),
            pl.BlockSpec((M2, bn), lambda n: (0, n)),
        ],
        out_specs=pl.BlockSpec((2, 1), lambda n: (0, 0)),
        out_shape=jax.ShapeDtypeStruct((2, 1), jnp.float32),
        scratch_shapes=[pltpu.VMEM((M, Na), jnp.float32)],
        compiler_params=pltpu.CompilerParams(
            dimension_semantics=("arbitrary",)),
    )(zst, wa, ba.reshape(1, Na), w, b.reshape(1, N), x, xn)


def _outhead_body(dq_ref, zq_ref, xe_ref, off_ref, w1_ref, b1_ref,
                  w2_ref, b2_ref, w3_ref, b3_ref,
                  out_ref, z1_ref, z2_ref, el_ref):
    zq = zq_ref[...]
    xe = xe_ref[...]
    d = zq - xe
    zst = xe + d
    dq = dq_ref[0] != 0
    zo = jnp.where(dq, zst, xe)
    M2 = zo.shape[0] // 2
    z1 = zo[:M2]
    z2 = zo[M2:]

    z1_ref[...] = z1
    z2_ref[...] = z2
    sq = d * d
    el_ref[...] = jnp.stack(
        [jnp.sum(sq[:M2]), jnp.sum(sq[M2:])]).reshape(2, 1)

    w1 = w1_ref[...]
    K = zo.shape[1]
    acc = jnp.dot(z1, w1[:K], preferred_element_type=jnp.float32)
    acc += jnp.dot(z2, w1[K:2 * K], preferred_element_type=jnp.float32)
    acc += jnp.dot(off_ref[...], w1[2 * K:],
                   preferred_element_type=jnp.float32)
    h1 = _leaky(acc + b1_ref[...])
    h2 = _leaky(jnp.dot(h1, w2_ref[...],
                        preferred_element_type=jnp.float32) + b2_ref[...])
    out_ref[...] = jnp.dot(h2, w3_ref[...],
                           preferred_element_type=jnp.float32) + b3_ref[...]


def _out_head(dq, zq0, xe, offs, w1, b1, w2, b2, w3, b3):
    M, K = zq0.shape
    M2 = M // 2
    K3, N1 = w1.shape
    _, N3 = w3.shape
    return pl.pallas_call(
        _outhead_body,
        in_specs=[
            pl.BlockSpec(memory_space=pltpu.SMEM),
            pl.BlockSpec((M, K), lambda: (0, 0)),
            pl.BlockSpec((M, K), lambda: (0, 0)),
            pl.BlockSpec((M2, K), lambda: (0, 0)),
            pl.BlockSpec((K3, N1), lambda: (0, 0)),
            pl.BlockSpec((1, N1), lambda: (0, 0)),
            pl.BlockSpec((N1, N1), lambda: (0, 0)),
            pl.BlockSpec((1, N1), lambda: (0, 0)),
            pl.BlockSpec((N1, N3), lambda: (0, 0)),
            pl.BlockSpec((1, N3), lambda: (0, 0)),
        ],
        out_specs=[
            pl.BlockSpec((M2, N3), lambda: (0, 0)),
            pl.BlockSpec((M2, K), lambda: (0, 0)),
            pl.BlockSpec((M2, K), lambda: (0, 0)),
            pl.BlockSpec((2, 1), lambda: (0, 0)),
        ],
        out_shape=[
            jax.ShapeDtypeStruct((M2, N3), jnp.float32),
            jax.ShapeDtypeStruct((M2, K), jnp.float32),
            jax.ShapeDtypeStruct((M2, K), jnp.float32),
            jax.ShapeDtypeStruct((2, 1), jnp.float32),
        ],
    )(dq, zq0, xe, offs, w1, b1.reshape(1, N1),
      w2, b2.reshape(1, N1), w3, b3.reshape(1, N3))


def _sc_mesh_info():
    info = plsc.get_sparse_core_info()
    return info.num_cores, info.num_subcores


def _gather_offset(otab, oidx):
    nc, ns = _sc_mesh_info()
    nw = nc * ns
    bO = oidx.shape[0] // nw
    mesh = plsc.VectorSubcoreMesh(core_axis_name="c", subcore_axis_name="s")

    @functools.partial(
        pl.kernel, mesh=mesh,
        out_type=jax.ShapeDtypeStruct((oidx.shape[0], otab.shape[1]),
                                      jnp.float32),
        scratch_types=[
            pltpu.VMEM((bO,), jnp.int32),
            pltpu.VMEM((bO, otab.shape[1]), jnp.float32),
            pltpu.SemaphoreType.DMA,
        ],
    )
    def k(otab_hbm, oidx_hbm, offs_hbm, oidx_v, orows_v, sem_o):
        wid = lax.axis_index("s") * nc + lax.axis_index("c")
        obase = wid * bO
        pltpu.sync_copy(oidx_hbm.at[pl.ds(obase, bO)], oidx_v)
        pltpu.async_copy(otab_hbm.at[oidx_v], orows_v, sem_o).wait()
        pltpu.sync_copy(orows_v, offs_hbm.at[pl.ds(obase, bO)])

    return k(otab, oidx)


def _gather_q0(tab, idx, n_chunks=4):
    nc, ns = _sc_mesh_info()
    nw = nc * ns
    bq = idx.shape[0] // nw
    ck = bq // n_chunks
    mesh = plsc.VectorSubcoreMesh(core_axis_name="c", subcore_axis_name="s")

    @functools.partial(
        pl.kernel, mesh=mesh,
        out_type=jax.ShapeDtypeStruct((idx.shape[0], tab.shape[1]),
                                      jnp.float32),
        scratch_types=[
            pltpu.VMEM((bq,), jnp.int32),
            pltpu.VMEM((bq, tab.shape[1]), jnp.float32),
            pltpu.SemaphoreType.DMA,
        ],
    )
    def k(tab_hbm, idx_hbm, out_hbm, idx_v, rows_v, sem):
        wid = lax.axis_index("s") * nc + lax.axis_index("c")
        base = wid * bq
        pltpu.sync_copy(idx_hbm.at[pl.ds(base, bq)], idx_v)
        cps = []
        for ch in range(n_chunks):
            cps.append(pltpu.async_copy(
                tab_hbm.at[idx_v.at[pl.ds(ch * ck, ck)]],
                rows_v.at[pl.ds(ch * ck, ck)], sem))
        for cp in cps:
            cp.wait()
        pltpu.sync_copy(rows_v, out_hbm.at[pl.ds(base, bq)])

    return k(tab, idx)


def kernel(x, x_next, k_offset, do_quantize, k, params):
    p = params
    B = x.shape[0]
    xf1 = x.reshape(B, -1)
    xf2 = x_next.reshape(B, -1)

    koff = k_offset.astype(jnp.int32)
    offs = _gather_offset(p['offset_table'], koff)

    zenc = _ae_mlp(xf1, xf2, p['ae_W1'], p['ae_b1'],
                   p['ae_W2'], p['ae_b2'], p['ae_W3'], p['ae_b3'])

    zst_ae, diff_ae = _ae_vq(zenc, p['ae_q_embed'])

    q0 = p['q0_embed'].reshape(8192, 512)
    xe, ind = _enc_vq(zst_ae, p['enc_W1'], p['enc_b1'],
                      p['enc_W2'], p['enc_b2'], q0)
    zq0 = _gather_q0(q0, ind.reshape(-1))

    rec_ss = _dec_loss(zst_ae, p['aed_W1'], p['aed_b1'],
                       p['aed_W2'], p['aed_b2'], xf1, xf2)

    nae = zenc.shape[0] // 2
    den_z = nae * zenc.shape[1]
    den_x = nae * xf1.shape[1]
    ae_loss_1 = rec_ss[0, 0] / den_x * 10.0 + diff_ae[0, 0] / den_z
    ae_loss_2 = rec_ss[1, 0] / den_x * 10.0 + diff_ae[1, 0] / den_z

    dq_arr = jnp.asarray(do_quantize, jnp.int32).reshape(1)
    w3 = jnp.pad(p['out_W3'], ((0, 0), (0, 118)))
    b3 = jnp.pad(p['out_b3'], (0, 118))
    out, z1, z2, el_ss = _out_head(dq_arr, zq0, xe, offs,
                                   p['out_W1'], p['out_b1'],
                                   p['out_W2'], p['out_b2'], w3, b3)
    out = out[:, :10]

    dq = do_quantize != 0
    el_1 = jnp.where(dq, el_ss[0, 0] / den_z, jnp.float32(0.0))
    el_2 = jnp.where(dq, el_ss[1, 0] / den_z, jnp.float32(0.0))

    loss = ae_loss_1 + ae_loss_2 + el_1 + el_2
    ind_1 = ind[:nae]
    ind_2 = ind[nae:]
    return (out, loss, ind_1, ind_2, z1, z2)
